# Initial kernel scaffold; baseline (speedup 1.0000x reference)
import jax
import jax.numpy as jnp
from jax.experimental import pallas as pl


def kernel(x, adj_low, adj_high, adj_nd_low, adj_nd_high, conv0_W, conv0_b, conv1_W, conv1_b, bn_gamma, bn_beta, bn_mean, bn_var, fp_W, fp_b):
    raise NotImplementedError("write your pallas kernel here")


# speedup vs baseline: 1.4843x; 1.4843x over previous
import jax
import jax.numpy as jnp
from jax.experimental import pallas as pl

_N = 10000
_BM = 200
_BL = 2000

_f32 = jnp.float32
_bf16 = jnp.bfloat16


def _lin_kernel(x_ref, wT_ref, b_ref, p0_ref, p12_ref):
    p = jnp.dot(x_ref[...], wT_ref[...], preferred_element_type=_f32) + b_ref[...]
    p0_ref[...] = p[:, :64]
    p12_ref[...] = p[:, 64:]


def _hop1_kernel(al_ref, an_ref, p12_ref, y1_ref, tl_ref, tn_ref):
    al = al_ref[...].astype(_bf16)
    an = an_ref[...].astype(_bf16)
    p12 = p12_ref[...].astype(_bf16)
    u = jnp.dot(al, p12, preferred_element_type=_f32)
    v = jnp.dot(an, p12, preferred_element_type=_f32)
    y1_ref[...] = u[:, :64] + 0.5 * v[:, :64]
    tl_ref[...] = u[:, 64:]
    tn_ref[...] = v[:, 64:]


def _hop2_l0_kernel(al_ref, an_ref, tl_ref, tn_ref, p0_ref, y1_ref,
                    wT_ref, b_ref, sc_ref, sh_ref, q0_ref, q12_ref):
    al = al_ref[...].astype(_bf16)
    an = an_ref[...].astype(_bf16)
    tl = tl_ref[...].astype(_bf16)
    tn = tn_ref[...].astype(_bf16)
    y2 = (jnp.dot(al, tl, preferred_element_type=_f32)
          + 0.5 * jnp.dot(an, tn, preferred_element_type=_f32))
    h = jnp.concatenate([1.5 * p0_ref[...], y1_ref[...], y2], axis=1)
    h = h * sc_ref[...] + sh_ref[...]
    h = jnp.maximum(h, 0.0)
    q = jnp.dot(h, wT_ref[...], preferred_element_type=_f32) + b_ref[...]
    q0_ref[...] = q[:, :64]
    q12_ref[...] = q[:, 64:]


def _hop2_l1_kernel(al_ref, an_ref, tl_ref, tn_ref, q0_ref, y1_ref,
                    wT_ref, b_ref, o_ref):
    al = al_ref[...].astype(_bf16)
    an = an_ref[...].astype(_bf16)
    tl = tl_ref[...].astype(_bf16)
    tn = tn_ref[...].astype(_bf16)
    y2 = (jnp.dot(al, tl, preferred_element_type=_f32)
          + 0.5 * jnp.dot(an, tn, preferred_element_type=_f32))
    h = jnp.concatenate([1.5 * q0_ref[...], y1_ref[...], y2], axis=1)
    o_ref[...] = jnp.dot(h, wT_ref[...], preferred_element_type=_f32) + b_ref[...]


def _sweep1(a_low, a_nd, p12):
    return pl.pallas_call(
        _hop1_kernel,
        grid=(_N // _BM,),
        in_specs=[pl.BlockSpec((_BM, _N), lambda i: (i, 0)),
                  pl.BlockSpec((_BM, _N), lambda i: (i, 0)),
                  pl.BlockSpec((_N, 128), lambda i: (0, 0))],
        out_specs=[pl.BlockSpec((_BM, 64), lambda i: (i, 0)),
                   pl.BlockSpec((_BM, 64), lambda i: (i, 0)),
                   pl.BlockSpec((_BM, 64), lambda i: (i, 0))],
        out_shape=[jax.ShapeDtypeStruct((_N, 64), _f32),
                   jax.ShapeDtypeStruct((_N, 64), _f32),
                   jax.ShapeDtypeStruct((_N, 64), _f32)],
    )(a_low, a_nd, p12)


def kernel(x, adj_low, adj_high, adj_nd_low, adj_nd_high,
           conv0_W, conv0_b, conv1_W, conv1_b,
           bn_gamma, bn_beta, bn_mean, bn_var, fp_W, fp_b):
    w0T = conv0_W.reshape(192, 128).T
    b0 = conv0_b.reshape(1, 192)
    w1T = conv1_W.reshape(192, 192).T
    b1 = conv1_b.reshape(1, 192)
    fpT = fp_W.T
    fpb = fp_b.reshape(1, 64)
    inv = jax.lax.rsqrt(bn_var + 1e-5)
    bn_sc = (bn_gamma * inv).reshape(1, 192)
    bn_sh = (bn_beta - bn_mean * bn_gamma * inv).reshape(1, 192)

    p0, p12 = pl.pallas_call(
        _lin_kernel,
        grid=(_N // _BL,),
        in_specs=[pl.BlockSpec((_BL, 128), lambda i: (i, 0)),
                  pl.BlockSpec((128, 192), lambda i: (0, 0)),
                  pl.BlockSpec((1, 192), lambda i: (0, 0))],
        out_specs=[pl.BlockSpec((_BL, 64), lambda i: (i, 0)),
                   pl.BlockSpec((_BL, 128), lambda i: (i, 0))],
        out_shape=[jax.ShapeDtypeStruct((_N, 64), _f32),
                   jax.ShapeDtypeStruct((_N, 128), _f32)],
    )(x, w0T, b0)

    y1, tl, tn = _sweep1(adj_low, adj_nd_low, p12)

    q0, q12 = pl.pallas_call(
        _hop2_l0_kernel,
        grid=(_N // _BM,),
        in_specs=[pl.BlockSpec((_BM, _N), lambda i: (i, 0)),
                  pl.BlockSpec((_BM, _N), lambda i: (i, 0)),
                  pl.BlockSpec((_N, 64), lambda i: (0, 0)),
                  pl.BlockSpec((_N, 64), lambda i: (0, 0)),
                  pl.BlockSpec((_BM, 64), lambda i: (i, 0)),
                  pl.BlockSpec((_BM, 64), lambda i: (i, 0)),
                  pl.BlockSpec((192, 192), lambda i: (0, 0)),
                  pl.BlockSpec((1, 192), lambda i: (0, 0)),
                  pl.BlockSpec((1, 192), lambda i: (0, 0)),
                  pl.BlockSpec((1, 192), lambda i: (0, 0))],
        out_specs=[pl.BlockSpec((_BM, 64), lambda i: (i, 0)),
                   pl.BlockSpec((_BM, 128), lambda i: (i, 0))],
        out_shape=[jax.ShapeDtypeStruct((_N, 64), _f32),
                   jax.ShapeDtypeStruct((_N, 128), _f32)],
    )(adj_low, adj_nd_low, tl, tn, p0, y1, w1T, b1, bn_sc, bn_sh)

    z1, sl, sn = _sweep1(adj_low, adj_nd_low, q12)

    out = pl.pallas_call(
        _hop2_l1_kernel,
        grid=(_N // _BM,),
        in_specs=[pl.BlockSpec((_BM, _N), lambda i: (i, 0)),
                  pl.BlockSpec((_BM, _N), lambda i: (i, 0)),
                  pl.BlockSpec((_N, 64), lambda i: (0, 0)),
                  pl.BlockSpec((_N, 64), lambda i: (0, 0)),
                  pl.BlockSpec((_BM, 64), lambda i: (i, 0)),
                  pl.BlockSpec((_BM, 64), lambda i: (i, 0)),
                  pl.BlockSpec((192, 64), lambda i: (0, 0)),
                  pl.BlockSpec((1, 64), lambda i: (0, 0))],
        out_specs=pl.BlockSpec((_BM, 64), lambda i: (i, 0)),
        out_shape=jax.ShapeDtypeStruct((_N, 64), _f32),
    )(adj_low, adj_nd_low, sl, sn, q0, z1, fpT, fpb)

    return out


# speedup vs baseline: 1.8065x; 1.2171x over previous
import jax
import jax.numpy as jnp
from jax.experimental import pallas as pl

_N = 10000
_BM1 = 80
_BM2 = 400
_BL = 2000

_f32 = jnp.float32
_bf16 = jnp.bfloat16


def _lin_kernel(x_ref, wT_ref, b_ref, p0_ref, p12_ref):
    p = jnp.dot(x_ref[...], wT_ref[...], preferred_element_type=_f32) + b_ref[...]
    p0_ref[...] = p[:, :64]
    p12_ref[...] = p[:, 64:]


def _hop1_cache_kernel(al_ref, an_ref, p12_ref,
                       y1_ref, tl_ref, tn_ref, albf_ref, anbf_ref):
    al = al_ref[...].astype(_bf16)
    an = an_ref[...].astype(_bf16)
    albf_ref[...] = al
    anbf_ref[...] = an
    p12 = p12_ref[...].astype(_bf16)
    u = jnp.dot(al, p12, preferred_element_type=_f32)
    v = jnp.dot(an, p12, preferred_element_type=_f32)
    y1_ref[...] = u[:, :64] + 0.5 * v[:, :64]
    tl_ref[...] = u[:, 64:]
    tn_ref[...] = v[:, 64:]


def _hop1_kernel(al_ref, an_ref, p12_ref, y1_ref, tl_ref, tn_ref):
    p12 = p12_ref[...].astype(_bf16)
    u = jnp.dot(al_ref[...], p12, preferred_element_type=_f32)
    v = jnp.dot(an_ref[...], p12, preferred_element_type=_f32)
    y1_ref[...] = u[:, :64] + 0.5 * v[:, :64]
    tl_ref[...] = u[:, 64:]
    tn_ref[...] = v[:, 64:]


def _hop2_l0_kernel(al_ref, an_ref, tl_ref, tn_ref, p0_ref, y1_ref,
                    wT_ref, b_ref, sc_ref, sh_ref, q0_ref, q12_ref):
    tl = tl_ref[...].astype(_bf16)
    tn = tn_ref[...].astype(_bf16)
    y2 = (jnp.dot(al_ref[...], tl, preferred_element_type=_f32)
          + 0.5 * jnp.dot(an_ref[...], tn, preferred_element_type=_f32))
    h = jnp.concatenate([1.5 * p0_ref[...], y1_ref[...], y2], axis=1)
    h = h * sc_ref[...] + sh_ref[...]
    h = jnp.maximum(h, 0.0)
    q = jnp.dot(h, wT_ref[...], preferred_element_type=_f32) + b_ref[...]
    q0_ref[...] = q[:, :64]
    q12_ref[...] = q[:, 64:]


def _hop2_l1_kernel(al_ref, an_ref, tl_ref, tn_ref, q0_ref, y1_ref,
                    wT_ref, b_ref, o_ref):
    tl = tl_ref[...].astype(_bf16)
    tn = tn_ref[...].astype(_bf16)
    y2 = (jnp.dot(al_ref[...], tl, preferred_element_type=_f32)
          + 0.5 * jnp.dot(an_ref[...], tn, preferred_element_type=_f32))
    h = jnp.concatenate([1.5 * q0_ref[...], y1_ref[...], y2], axis=1)
    o_ref[...] = jnp.dot(h, wT_ref[...], preferred_element_type=_f32) + b_ref[...]


def kernel(x, adj_low, adj_high, adj_nd_low, adj_nd_high,
           conv0_W, conv0_b, conv1_W, conv1_b,
           bn_gamma, bn_beta, bn_mean, bn_var, fp_W, fp_b):
    w0T = conv0_W.reshape(192, 128).T
    b0 = conv0_b.reshape(1, 192)
    w1T = conv1_W.reshape(192, 192).T
    b1 = conv1_b.reshape(1, 192)
    fpT = fp_W.T
    fpb = fp_b.reshape(1, 64)
    inv = jax.lax.rsqrt(bn_var + 1e-5)
    bn_sc = (bn_gamma * inv).reshape(1, 192)
    bn_sh = (bn_beta - bn_mean * bn_gamma * inv).reshape(1, 192)

    p0, p12 = pl.pallas_call(
        _lin_kernel,
        grid=(_N // _BL,),
        in_specs=[pl.BlockSpec((_BL, 128), lambda i: (i, 0)),
                  pl.BlockSpec((128, 192), lambda i: (0, 0)),
                  pl.BlockSpec((1, 192), lambda i: (0, 0))],
        out_specs=[pl.BlockSpec((_BL, 64), lambda i: (i, 0)),
                   pl.BlockSpec((_BL, 128), lambda i: (i, 0))],
        out_shape=[jax.ShapeDtypeStruct((_N, 64), _f32),
                   jax.ShapeDtypeStruct((_N, 128), _f32)],
    )(x, w0T, b0)

    y1, tl, tn, albf, anbf = pl.pallas_call(
        _hop1_cache_kernel,
        grid=(_N // _BM1,),
        in_specs=[pl.BlockSpec((_BM1, _N), lambda i: (i, 0)),
                  pl.BlockSpec((_BM1, _N), lambda i: (i, 0)),
                  pl.BlockSpec((_N, 128), lambda i: (0, 0))],
        out_specs=[pl.BlockSpec((_BM1, 64), lambda i: (i, 0)),
                   pl.BlockSpec((_BM1, 64), lambda i: (i, 0)),
                   pl.BlockSpec((_BM1, 64), lambda i: (i, 0)),
                   pl.BlockSpec((_BM1, _N), lambda i: (i, 0)),
                   pl.BlockSpec((_BM1, _N), lambda i: (i, 0))],
        out_shape=[jax.ShapeDtypeStruct((_N, 64), _f32),
                   jax.ShapeDtypeStruct((_N, 64), _f32),
                   jax.ShapeDtypeStruct((_N, 64), _f32),
                   jax.ShapeDtypeStruct((_N, _N), _bf16),
                   jax.ShapeDtypeStruct((_N, _N), _bf16)],
    )(adj_low, adj_nd_low, p12)

    q0, q12 = pl.pallas_call(
        _hop2_l0_kernel,
        grid=(_N // _BM2,),
        in_specs=[pl.BlockSpec((_BM2, _N), lambda i: (i, 0)),
                  pl.BlockSpec((_BM2, _N), lambda i: (i, 0)),
                  pl.BlockSpec((_N, 64), lambda i: (0, 0)),
                  pl.BlockSpec((_N, 64), lambda i: (0, 0)),
                  pl.BlockSpec((_BM2, 64), lambda i: (i, 0)),
                  pl.BlockSpec((_BM2, 64), lambda i: (i, 0)),
                  pl.BlockSpec((192, 192), lambda i: (0, 0)),
                  pl.BlockSpec((1, 192), lambda i: (0, 0)),
                  pl.BlockSpec((1, 192), lambda i: (0, 0)),
                  pl.BlockSpec((1, 192), lambda i: (0, 0))],
        out_specs=[pl.BlockSpec((_BM2, 64), lambda i: (i, 0)),
                   pl.BlockSpec((_BM2, 128), lambda i: (i, 0))],
        out_shape=[jax.ShapeDtypeStruct((_N, 64), _f32),
                   jax.ShapeDtypeStruct((_N, 128), _f32)],
    )(albf, anbf, tl, tn, p0, y1, w1T, b1, bn_sc, bn_sh)

    z1, sl, sn = pl.pallas_call(
        _hop1_kernel,
        grid=(_N // _BM2,),
        in_specs=[pl.BlockSpec((_BM2, _N), lambda i: (i, 0)),
                  pl.BlockSpec((_BM2, _N), lambda i: (i, 0)),
                  pl.BlockSpec((_N, 128), lambda i: (0, 0))],
        out_specs=[pl.BlockSpec((_BM2, 64), lambda i: (i, 0)),
                   pl.BlockSpec((_BM2, 64), lambda i: (i, 0)),
                   pl.BlockSpec((_BM2, 64), lambda i: (i, 0))],
        out_shape=[jax.ShapeDtypeStruct((_N, 64), _f32),
                   jax.ShapeDtypeStruct((_N, 64), _f32),
                   jax.ShapeDtypeStruct((_N, 64), _f32)],
    )(albf, anbf, q12)

    out = pl.pallas_call(
        _hop2_l1_kernel,
        grid=(_N // _BM2,),
        in_specs=[pl.BlockSpec((_BM2, _N), lambda i: (i, 0)),
                  pl.BlockSpec((_BM2, _N), lambda i: (i, 0)),
                  pl.BlockSpec((_N, 64), lambda i: (0, 0)),
                  pl.BlockSpec((_N, 64), lambda i: (0, 0)),
                  pl.BlockSpec((_BM2, 64), lambda i: (i, 0)),
                  pl.BlockSpec((_BM2, 64), lambda i: (i, 0)),
                  pl.BlockSpec((192, 64), lambda i: (0, 0)),
                  pl.BlockSpec((1, 64), lambda i: (0, 0))],
        out_specs=pl.BlockSpec((_BM2, 64), lambda i: (i, 0)),
        out_shape=jax.ShapeDtypeStruct((_N, 64), _f32),
    )(albf, anbf, sl, sn, q0, z1, fpT, fpb)

    return out


# speedup vs baseline: 1.8845x; 1.0432x over previous
import jax
import jax.numpy as jnp
from jax.experimental import pallas as pl

_N = 10000
_BM1 = 160
_BM2 = 512
_BL = 2000

_f32 = jnp.float32
_bf16 = jnp.bfloat16


def _lin_kernel(x_ref, wT_ref, b_ref, p0_ref, p12_ref):
    p = jnp.dot(x_ref[...], wT_ref[...], preferred_element_type=_f32) + b_ref[...]
    p0_ref[...] = p[:, :64]
    p12_ref[...] = p[:, 64:].astype(_bf16)


def _hop1_cache_kernel(al_ref, an_ref, p12_ref,
                       y1_ref, tl_ref, tn_ref, albf_ref, anbf_ref):
    al = al_ref[...].astype(_bf16)
    an = an_ref[...].astype(_bf16)
    albf_ref[...] = al
    anbf_ref[...] = an
    p12 = p12_ref[...]
    u = jnp.dot(al, p12, preferred_element_type=_f32)
    v = jnp.dot(an, p12, preferred_element_type=_f32)
    y1_ref[...] = u[:, :64] + 0.5 * v[:, :64]
    tl_ref[...] = u[:, 64:].astype(_bf16)
    tn_ref[...] = v[:, 64:].astype(_bf16)


def _hop1_kernel(al_ref, an_ref, p12_ref, y1_ref, tl_ref, tn_ref):
    p12 = p12_ref[...]
    u = jnp.dot(al_ref[...], p12, preferred_element_type=_f32)
    v = jnp.dot(an_ref[...], p12, preferred_element_type=_f32)
    y1_ref[...] = u[:, :64] + 0.5 * v[:, :64]
    tl_ref[...] = u[:, 64:].astype(_bf16)
    tn_ref[...] = v[:, 64:].astype(_bf16)


def _hop2_l0_kernel(al_ref, an_ref, tl_ref, tn_ref, p0_ref, y1_ref,
                    wT_ref, b_ref, sc_ref, sh_ref, q0_ref, q12_ref):
    y2 = (jnp.dot(al_ref[...], tl_ref[...], preferred_element_type=_f32)
          + 0.5 * jnp.dot(an_ref[...], tn_ref[...], preferred_element_type=_f32))
    h = jnp.concatenate([1.5 * p0_ref[...], y1_ref[...], y2], axis=1)
    h = h * sc_ref[...] + sh_ref[...]
    h = jnp.maximum(h, 0.0)
    q = jnp.dot(h, wT_ref[...], preferred_element_type=_f32) + b_ref[...]
    q0_ref[...] = q[:, :64]
    q12_ref[...] = q[:, 64:].astype(_bf16)


def _hop2_l1_kernel(al_ref, an_ref, tl_ref, tn_ref, q0_ref, y1_ref,
                    wT_ref, b_ref, o_ref):
    y2 = (jnp.dot(al_ref[...], tl_ref[...], preferred_element_type=_f32)
          + 0.5 * jnp.dot(an_ref[...], tn_ref[...], preferred_element_type=_f32))
    h = jnp.concatenate([1.5 * q0_ref[...], y1_ref[...], y2], axis=1)
    o_ref[...] = jnp.dot(h, wT_ref[...], preferred_element_type=_f32) + b_ref[...]


def kernel(x, adj_low, adj_high, adj_nd_low, adj_nd_high,
           conv0_W, conv0_b, conv1_W, conv1_b,
           bn_gamma, bn_beta, bn_mean, bn_var, fp_W, fp_b):
    w0T = conv0_W.reshape(192, 128).T
    b0 = conv0_b.reshape(1, 192)
    w1T = conv1_W.reshape(192, 192).T
    b1 = conv1_b.reshape(1, 192)
    fpT = fp_W.T
    fpb = fp_b.reshape(1, 64)
    inv = jax.lax.rsqrt(bn_var + 1e-5)
    bn_sc = (bn_gamma * inv).reshape(1, 192)
    bn_sh = (bn_beta - bn_mean * bn_gamma * inv).reshape(1, 192)

    p0, p12 = pl.pallas_call(
        _lin_kernel,
        grid=(_N // _BL,),
        in_specs=[pl.BlockSpec((_BL, 128), lambda i: (i, 0)),
                  pl.BlockSpec((128, 192), lambda i: (0, 0)),
                  pl.BlockSpec((1, 192), lambda i: (0, 0))],
        out_specs=[pl.BlockSpec((_BL, 64), lambda i: (i, 0)),
                   pl.BlockSpec((_BL, 128), lambda i: (i, 0))],
        out_shape=[jax.ShapeDtypeStruct((_N, 64), _f32),
                   jax.ShapeDtypeStruct((_N, 128), _bf16)],
    )(x, w0T, b0)

    ng1 = (_N + _BM1 - 1) // _BM1
    y1, tl, tn, albf, anbf = pl.pallas_call(
        _hop1_cache_kernel,
        grid=(ng1,),
        in_specs=[pl.BlockSpec((_BM1, _N), lambda i: (i, 0)),
                  pl.BlockSpec((_BM1, _N), lambda i: (i, 0)),
                  pl.BlockSpec((_N, 128), lambda i: (0, 0))],
        out_specs=[pl.BlockSpec((_BM1, 64), lambda i: (i, 0)),
                   pl.BlockSpec((_BM1, 64), lambda i: (i, 0)),
                   pl.BlockSpec((_BM1, 64), lambda i: (i, 0)),
                   pl.BlockSpec((_BM1, _N), lambda i: (i, 0)),
                   pl.BlockSpec((_BM1, _N), lambda i: (i, 0))],
        out_shape=[jax.ShapeDtypeStruct((_N, 64), _f32),
                   jax.ShapeDtypeStruct((_N, 64), _bf16),
                   jax.ShapeDtypeStruct((_N, 64), _bf16),
                   jax.ShapeDtypeStruct((_N, _N), _bf16),
                   jax.ShapeDtypeStruct((_N, _N), _bf16)],
    )(adj_low, adj_nd_low, p12)

    ng2 = (_N + _BM2 - 1) // _BM2

    q0, q12 = pl.pallas_call(
        _hop2_l0_kernel,
        grid=(ng2,),
        in_specs=[pl.BlockSpec((_BM2, _N), lambda i: (i, 0)),
                  pl.BlockSpec((_BM2, _N), lambda i: (i, 0)),
                  pl.BlockSpec((_N, 64), lambda i: (0, 0)),
                  pl.BlockSpec((_N, 64), lambda i: (0, 0)),
                  pl.BlockSpec((_BM2, 64), lambda i: (i, 0)),
                  pl.BlockSpec((_BM2, 64), lambda i: (i, 0)),
                  pl.BlockSpec((192, 192), lambda i: (0, 0)),
                  pl.BlockSpec((1, 192), lambda i: (0, 0)),
                  pl.BlockSpec((1, 192), lambda i: (0, 0)),
                  pl.BlockSpec((1, 192), lambda i: (0, 0))],
        out_specs=[pl.BlockSpec((_BM2, 64), lambda i: (i, 0)),
                   pl.BlockSpec((_BM2, 128), lambda i: (i, 0))],
        out_shape=[jax.ShapeDtypeStruct((_N, 64), _f32),
                   jax.ShapeDtypeStruct((_N, 128), _bf16)],
    )(albf, anbf, tl, tn, p0, y1, w1T, b1, bn_sc, bn_sh)

    z1, sl, sn = pl.pallas_call(
        _hop1_kernel,
        grid=(ng2,),
        in_specs=[pl.BlockSpec((_BM2, _N), lambda i: (i, 0)),
                  pl.BlockSpec((_BM2, _N), lambda i: (i, 0)),
                  pl.BlockSpec((_N, 128), lambda i: (0, 0))],
        out_specs=[pl.BlockSpec((_BM2, 64), lambda i: (i, 0)),
                   pl.BlockSpec((_BM2, 64), lambda i: (i, 0)),
                   pl.BlockSpec((_BM2, 64), lambda i: (i, 0))],
        out_shape=[jax.ShapeDtypeStruct((_N, 64), _f32),
                   jax.ShapeDtypeStruct((_N, 64), _bf16),
                   jax.ShapeDtypeStruct((_N, 64), _bf16)],
    )(albf, anbf, q12)

    out = pl.pallas_call(
        _hop2_l1_kernel,
        grid=(ng2,),
        in_specs=[pl.BlockSpec((_BM2, _N), lambda i: (i, 0)),
                  pl.BlockSpec((_BM2, _N), lambda i: (i, 0)),
                  pl.BlockSpec((_N, 64), lambda i: (0, 0)),
                  pl.BlockSpec((_N, 64), lambda i: (0, 0)),
                  pl.BlockSpec((_BM2, 64), lambda i: (i, 0)),
                  pl.BlockSpec((_BM2, 64), lambda i: (i, 0)),
                  pl.BlockSpec((192, 64), lambda i: (0, 0)),
                  pl.BlockSpec((1, 64), lambda i: (0, 0))],
        out_specs=pl.BlockSpec((_BM2, 64), lambda i: (i, 0)),
        out_shape=jax.ShapeDtypeStruct((_N, 64), _f32),
    )(albf, anbf, sl, sn, q0, z1, fpT, fpb)

    return out


# speedup vs baseline: 1.8880x; 1.0019x over previous
import jax
import jax.numpy as jnp
from jax.experimental import pallas as pl

_N = 10000
_BM1 = 192
_BM2 = 640
_BL = 2000

_f32 = jnp.float32
_bf16 = jnp.bfloat16


def _lin_kernel(x_ref, wT_ref, b_ref, p0_ref, p12_ref):
    p = jnp.dot(x_ref[...], wT_ref[...], preferred_element_type=_f32) + b_ref[...]
    p0_ref[...] = p[:, :64]
    p12_ref[...] = p[:, 64:].astype(_bf16)


def _hop1_cache_kernel(al_ref, an_ref, p12_ref,
                       y1_ref, tl_ref, tn_ref, albf_ref, anbf_ref):
    al = al_ref[...].astype(_bf16)
    an = an_ref[...].astype(_bf16)
    albf_ref[...] = al
    anbf_ref[...] = an
    p12 = p12_ref[...]
    u = jnp.dot(al, p12, preferred_element_type=_f32)
    v = jnp.dot(an, p12, preferred_element_type=_f32)
    y1_ref[...] = u[:, :64] + 0.5 * v[:, :64]
    tl_ref[...] = u[:, 64:].astype(_bf16)
    tn_ref[...] = v[:, 64:].astype(_bf16)


def _hop1_kernel(al_ref, an_ref, p12_ref, y1_ref, tl_ref, tn_ref):
    p12 = p12_ref[...]
    u = jnp.dot(al_ref[...], p12, preferred_element_type=_f32)
    v = jnp.dot(an_ref[...], p12, preferred_element_type=_f32)
    y1_ref[...] = u[:, :64] + 0.5 * v[:, :64]
    tl_ref[...] = u[:, 64:].astype(_bf16)
    tn_ref[...] = v[:, 64:].astype(_bf16)


def _hop2_l0_kernel(al_ref, an_ref, tl_ref, tn_ref, p0_ref, y1_ref,
                    wT_ref, b_ref, sc_ref, sh_ref, q0_ref, q12_ref):
    y2 = (jnp.dot(al_ref[...], tl_ref[...], preferred_element_type=_f32)
          + 0.5 * jnp.dot(an_ref[...], tn_ref[...], preferred_element_type=_f32))
    h = jnp.concatenate([1.5 * p0_ref[...], y1_ref[...], y2], axis=1)
    h = h * sc_ref[...] + sh_ref[...]
    h = jnp.maximum(h, 0.0)
    q = jnp.dot(h, wT_ref[...], preferred_element_type=_f32) + b_ref[...]
    q0_ref[...] = q[:, :64]
    q12_ref[...] = q[:, 64:].astype(_bf16)


def _hop2_l1_kernel(al_ref, an_ref, tl_ref, tn_ref, q0_ref, y1_ref,
                    wT_ref, b_ref, o_ref):
    y2 = (jnp.dot(al_ref[...], tl_ref[...], preferred_element_type=_f32)
          + 0.5 * jnp.dot(an_ref[...], tn_ref[...], preferred_element_type=_f32))
    h = jnp.concatenate([1.5 * q0_ref[...], y1_ref[...], y2], axis=1)
    o_ref[...] = jnp.dot(h, wT_ref[...], preferred_element_type=_f32) + b_ref[...]


def kernel(x, adj_low, adj_high, adj_nd_low, adj_nd_high,
           conv0_W, conv0_b, conv1_W, conv1_b,
           bn_gamma, bn_beta, bn_mean, bn_var, fp_W, fp_b):
    w0T = conv0_W.reshape(192, 128).T
    b0 = conv0_b.reshape(1, 192)
    w1T = conv1_W.reshape(192, 192).T
    b1 = conv1_b.reshape(1, 192)
    fpT = fp_W.T
    fpb = fp_b.reshape(1, 64)
    inv = jax.lax.rsqrt(bn_var + 1e-5)
    bn_sc = (bn_gamma * inv).reshape(1, 192)
    bn_sh = (bn_beta - bn_mean * bn_gamma * inv).reshape(1, 192)

    p0, p12 = pl.pallas_call(
        _lin_kernel,
        grid=(_N // _BL,),
        in_specs=[pl.BlockSpec((_BL, 128), lambda i: (i, 0)),
                  pl.BlockSpec((128, 192), lambda i: (0, 0)),
                  pl.BlockSpec((1, 192), lambda i: (0, 0))],
        out_specs=[pl.BlockSpec((_BL, 64), lambda i: (i, 0)),
                   pl.BlockSpec((_BL, 128), lambda i: (i, 0))],
        out_shape=[jax.ShapeDtypeStruct((_N, 64), _f32),
                   jax.ShapeDtypeStruct((_N, 128), _bf16)],
    )(x, w0T, b0)

    ng1 = (_N + _BM1 - 1) // _BM1
    y1, tl, tn, albf, anbf = pl.pallas_call(
        _hop1_cache_kernel,
        grid=(ng1,),
        in_specs=[pl.BlockSpec((_BM1, _N), lambda i: (i, 0)),
                  pl.BlockSpec((_BM1, _N), lambda i: (i, 0)),
                  pl.BlockSpec((_N, 128), lambda i: (0, 0))],
        out_specs=[pl.BlockSpec((_BM1, 64), lambda i: (i, 0)),
                   pl.BlockSpec((_BM1, 64), lambda i: (i, 0)),
                   pl.BlockSpec((_BM1, 64), lambda i: (i, 0)),
                   pl.BlockSpec((_BM1, _N), lambda i: (i, 0)),
                   pl.BlockSpec((_BM1, _N), lambda i: (i, 0))],
        out_shape=[jax.ShapeDtypeStruct((_N, 64), _f32),
                   jax.ShapeDtypeStruct((_N, 64), _bf16),
                   jax.ShapeDtypeStruct((_N, 64), _bf16),
                   jax.ShapeDtypeStruct((_N, _N), _bf16),
                   jax.ShapeDtypeStruct((_N, _N), _bf16)],
    )(adj_low, adj_nd_low, p12)

    ng2 = (_N + _BM2 - 1) // _BM2

    q0, q12 = pl.pallas_call(
        _hop2_l0_kernel,
        grid=(ng2,),
        in_specs=[pl.BlockSpec((_BM2, _N), lambda i: (i, 0)),
                  pl.BlockSpec((_BM2, _N), lambda i: (i, 0)),
                  pl.BlockSpec((_N, 64), lambda i: (0, 0)),
                  pl.BlockSpec((_N, 64), lambda i: (0, 0)),
                  pl.BlockSpec((_BM2, 64), lambda i: (i, 0)),
                  pl.BlockSpec((_BM2, 64), lambda i: (i, 0)),
                  pl.BlockSpec((192, 192), lambda i: (0, 0)),
                  pl.BlockSpec((1, 192), lambda i: (0, 0)),
                  pl.BlockSpec((1, 192), lambda i: (0, 0)),
                  pl.BlockSpec((1, 192), lambda i: (0, 0))],
        out_specs=[pl.BlockSpec((_BM2, 64), lambda i: (i, 0)),
                   pl.BlockSpec((_BM2, 128), lambda i: (i, 0))],
        out_shape=[jax.ShapeDtypeStruct((_N, 64), _f32),
                   jax.ShapeDtypeStruct((_N, 128), _bf16)],
    )(albf, anbf, tl, tn, p0, y1, w1T, b1, bn_sc, bn_sh)

    z1, sl, sn = pl.pallas_call(
        _hop1_kernel,
        grid=(ng2,),
        in_specs=[pl.BlockSpec((_BM2, _N), lambda i: (i, 0)),
                  pl.BlockSpec((_BM2, _N), lambda i: (i, 0)),
                  pl.BlockSpec((_N, 128), lambda i: (0, 0))],
        out_specs=[pl.BlockSpec((_BM2, 64), lambda i: (i, 0)),
                   pl.BlockSpec((_BM2, 64), lambda i: (i, 0)),
                   pl.BlockSpec((_BM2, 64), lambda i: (i, 0))],
        out_shape=[jax.ShapeDtypeStruct((_N, 64), _f32),
                   jax.ShapeDtypeStruct((_N, 64), _bf16),
                   jax.ShapeDtypeStruct((_N, 64), _bf16)],
    )(albf, anbf, q12)

    out = pl.pallas_call(
        _hop2_l1_kernel,
        grid=(ng2,),
        in_specs=[pl.BlockSpec((_BM2, _N), lambda i: (i, 0)),
                  pl.BlockSpec((_BM2, _N), lambda i: (i, 0)),
                  pl.BlockSpec((_N, 64), lambda i: (0, 0)),
                  pl.BlockSpec((_N, 64), lambda i: (0, 0)),
                  pl.BlockSpec((_BM2, 64), lambda i: (i, 0)),
                  pl.BlockSpec((_BM2, 64), lambda i: (i, 0)),
                  pl.BlockSpec((192, 64), lambda i: (0, 0)),
                  pl.BlockSpec((1, 64), lambda i: (0, 0))],
        out_specs=pl.BlockSpec((_BM2, 64), lambda i: (i, 0)),
        out_shape=jax.ShapeDtypeStruct((_N, 64), _f32),
    )(albf, anbf, sl, sn, q0, z1, fpT, fpb)

    return out


# speedup vs baseline: 1.8954x; 1.0039x over previous
import jax
import jax.numpy as jnp
from jax.experimental import pallas as pl
from jax.experimental.pallas import tpu as pltpu

_N = 10000
_BM1 = 160
_BM2 = 640

_f32 = jnp.float32
_bf16 = jnp.bfloat16


def _sweep1_fused_kernel(x_ref, wT_ref, b_ref, al_ref, an_ref,
                         y1_ref, tl_ref, tn_ref, albf_ref, anbf_ref, p0_ref,
                         p12_scr):
    i = pl.program_id(0)

    @pl.when(i == 0)
    def _lin_phase():
        p = jnp.dot(x_ref[...], wT_ref[...], preferred_element_type=_f32) + b_ref[...]
        p0_ref[...] = p[:, :64]
        p12_scr[...] = p[:, 64:].astype(_bf16)

    @pl.when(i > 0)
    def _sweep_phase():
        al = al_ref[...].astype(_bf16)
        an = an_ref[...].astype(_bf16)
        albf_ref[...] = al
        anbf_ref[...] = an
        p12 = p12_scr[...]
        u = jnp.dot(al, p12, preferred_element_type=_f32)
        v = jnp.dot(an, p12, preferred_element_type=_f32)
        y1_ref[...] = u[:, :64] + 0.5 * v[:, :64]
        tl_ref[...] = u[:, 64:].astype(_bf16)
        tn_ref[...] = v[:, 64:].astype(_bf16)


def _hop1_kernel(al_ref, an_ref, p12_ref, y1_ref, tl_ref, tn_ref):
    p12 = p12_ref[...]
    u = jnp.dot(al_ref[...], p12, preferred_element_type=_f32)
    v = jnp.dot(an_ref[...], p12, preferred_element_type=_f32)
    y1_ref[...] = u[:, :64] + 0.5 * v[:, :64]
    tl_ref[...] = u[:, 64:].astype(_bf16)
    tn_ref[...] = v[:, 64:].astype(_bf16)


def _hop2_l0_kernel(al_ref, an_ref, tl_ref, tn_ref, p0_ref, y1_ref,
                    wT_ref, b_ref, sc_ref, sh_ref, q0_ref, q12_ref):
    y2 = (jnp.dot(al_ref[...], tl_ref[...], preferred_element_type=_f32)
          + 0.5 * jnp.dot(an_ref[...], tn_ref[...], preferred_element_type=_f32))
    h = jnp.concatenate([1.5 * p0_ref[...], y1_ref[...], y2], axis=1)
    h = h * sc_ref[...] + sh_ref[...]
    h = jnp.maximum(h, 0.0)
    q = jnp.dot(h, wT_ref[...], preferred_element_type=_f32) + b_ref[...]
    q0_ref[...] = q[:, :64]
    q12_ref[...] = q[:, 64:].astype(_bf16)


def _hop2_l1_kernel(al_ref, an_ref, tl_ref, tn_ref, q0_ref, y1_ref,
                    wT_ref, b_ref, o_ref):
    y2 = (jnp.dot(al_ref[...], tl_ref[...], preferred_element_type=_f32)
          + 0.5 * jnp.dot(an_ref[...], tn_ref[...], preferred_element_type=_f32))
    h = jnp.concatenate([1.5 * q0_ref[...], y1_ref[...], y2], axis=1)
    o_ref[...] = jnp.dot(h, wT_ref[...], preferred_element_type=_f32) + b_ref[...]


def kernel(x, adj_low, adj_high, adj_nd_low, adj_nd_high,
           conv0_W, conv0_b, conv1_W, conv1_b,
           bn_gamma, bn_beta, bn_mean, bn_var, fp_W, fp_b):
    w0T = conv0_W.reshape(192, 128).T
    b0 = conv0_b.reshape(1, 192)
    w1T = conv1_W.reshape(192, 192).T
    b1 = conv1_b.reshape(1, 192)
    fpT = fp_W.T
    fpb = fp_b.reshape(1, 64)
    inv = jax.lax.rsqrt(bn_var + 1e-5)
    bn_sc = (bn_gamma * inv).reshape(1, 192)
    bn_sh = (bn_beta - bn_mean * bn_gamma * inv).reshape(1, 192)

    ng1 = (_N + _BM1 - 1) // _BM1 + 1
    _am = lambda i: (jnp.maximum(i - 1, 0), 0)
    y1, tl, tn, albf, anbf, p0 = pl.pallas_call(
        _sweep1_fused_kernel,
        grid=(ng1,),
        in_specs=[pl.BlockSpec((_N, 128), lambda i: (0, 0)),
                  pl.BlockSpec((128, 192), lambda i: (0, 0)),
                  pl.BlockSpec((1, 192), lambda i: (0, 0)),
                  pl.BlockSpec((_BM1, _N), _am),
                  pl.BlockSpec((_BM1, _N), _am)],
        out_specs=[pl.BlockSpec((_BM1, 64), _am),
                   pl.BlockSpec((_BM1, 64), _am),
                   pl.BlockSpec((_BM1, 64), _am),
                   pl.BlockSpec((_BM1, _N), _am),
                   pl.BlockSpec((_BM1, _N), _am),
                   pl.BlockSpec((_N, 64), lambda i: (0, 0))],
        out_shape=[jax.ShapeDtypeStruct((_N, 64), _f32),
                   jax.ShapeDtypeStruct((_N, 64), _bf16),
                   jax.ShapeDtypeStruct((_N, 64), _bf16),
                   jax.ShapeDtypeStruct((_N, _N), _bf16),
                   jax.ShapeDtypeStruct((_N, _N), _bf16),
                   jax.ShapeDtypeStruct((_N, 64), _f32)],
        scratch_shapes=[pltpu.VMEM((_N, 128), _bf16)],
    )(x, w0T, b0, adj_low, adj_nd_low)

    ng2 = (_N + _BM2 - 1) // _BM2

    q0, q12 = pl.pallas_call(
        _hop2_l0_kernel,
        grid=(ng2,),
        in_specs=[pl.BlockSpec((_BM2, _N), lambda i: (i, 0)),
                  pl.BlockSpec((_BM2, _N), lambda i: (i, 0)),
                  pl.BlockSpec((_N, 64), lambda i: (0, 0)),
                  pl.BlockSpec((_N, 64), lambda i: (0, 0)),
                  pl.BlockSpec((_BM2, 64), lambda i: (i, 0)),
                  pl.BlockSpec((_BM2, 64), lambda i: (i, 0)),
                  pl.BlockSpec((192, 192), lambda i: (0, 0)),
                  pl.BlockSpec((1, 192), lambda i: (0, 0)),
                  pl.BlockSpec((1, 192), lambda i: (0, 0)),
                  pl.BlockSpec((1, 192), lambda i: (0, 0))],
        out_specs=[pl.BlockSpec((_BM2, 64), lambda i: (i, 0)),
                   pl.BlockSpec((_BM2, 128), lambda i: (i, 0))],
        out_shape=[jax.ShapeDtypeStruct((_N, 64), _f32),
                   jax.ShapeDtypeStruct((_N, 128), _bf16)],
    )(albf, anbf, tl, tn, p0, y1, w1T, b1, bn_sc, bn_sh)

    z1, sl, sn = pl.pallas_call(
        _hop1_kernel,
        grid=(ng2,),
        in_specs=[pl.BlockSpec((_BM2, _N), lambda i: (i, 0)),
                  pl.BlockSpec((_BM2, _N), lambda i: (i, 0)),
                  pl.BlockSpec((_N, 128), lambda i: (0, 0))],
        out_specs=[pl.BlockSpec((_BM2, 64), lambda i: (i, 0)),
                   pl.BlockSpec((_BM2, 64), lambda i: (i, 0)),
                   pl.BlockSpec((_BM2, 64), lambda i: (i, 0))],
        out_shape=[jax.ShapeDtypeStruct((_N, 64), _f32),
                   jax.ShapeDtypeStruct((_N, 64), _bf16),
                   jax.ShapeDtypeStruct((_N, 64), _bf16)],
    )(albf, anbf, q12)

    out = pl.pallas_call(
        _hop2_l1_kernel,
        grid=(ng2,),
        in_specs=[pl.BlockSpec((_BM2, _N), lambda i: (i, 0)),
                  pl.BlockSpec((_BM2, _N), lambda i: (i, 0)),
                  pl.BlockSpec((_N, 64), lambda i: (0, 0)),
                  pl.BlockSpec((_N, 64), lambda i: (0, 0)),
                  pl.BlockSpec((_BM2, 64), lambda i: (i, 0)),
                  pl.BlockSpec((_BM2, 64), lambda i: (i, 0)),
                  pl.BlockSpec((192, 64), lambda i: (0, 0)),
                  pl.BlockSpec((1, 64), lambda i: (0, 0))],
        out_specs=pl.BlockSpec((_BM2, 64), lambda i: (i, 0)),
        out_shape=jax.ShapeDtypeStruct((_N, 64), _f32),
    )(albf, anbf, sl, sn, q0, z1, fpT, fpb)

    return out
